# baseline (speedup 1.0000x reference)
import functools

import jax
import jax.numpy as jnp
from jax import lax
from jax.experimental import pallas as pl
from jax.experimental.pallas import tpu as pltpu
from jax.experimental.pallas import tpu_sc as plsc

_NC, _NS, _L = 2, 16, 16
_NW = _NC * _NS


def _tc_add_body(a_ref, b_ref, o_ref):
    o_ref[...] = a_ref[...] + b_ref[...]


def _tc_add(x_1, x_2):
    M, N = x_1.shape
    BM = 4096
    return pl.pallas_call(
        _tc_add_body,
        out_shape=jax.ShapeDtypeStruct((M, N), x_1.dtype),
        grid=(M // BM,),
        in_specs=[
            pl.BlockSpec((BM, N), lambda i: (i, 0)),
            pl.BlockSpec((BM, N), lambda i: (i, 0)),
        ],
        out_specs=pl.BlockSpec((BM, N), lambda i: (i, 0)),
    )(x_1, x_2)


@functools.cache
def _sc_add(total, interpret=False):
    CB = 32768
    per_w = total // _NW
    chunks = per_w // CB
    mesh = plsc.VectorSubcoreMesh(
        core_axis_name="c", subcore_axis_name="s",
        num_cores=_NC, num_subcores=_NS)

    @functools.partial(
        pl.kernel,
        out_type=jax.ShapeDtypeStruct((total,), jnp.float32),
        mesh=mesh,
        scratch_types=[
            pltpu.VMEM((CB,), jnp.float32),
            pltpu.VMEM((CB,), jnp.float32),
        ],
        interpret=interpret,
    )
    def body(x1_hbm, x2_hbm, out_hbm, a_v, b_v):
        wid = lax.axis_index("s") * _NC + lax.axis_index("c")
        base = wid * per_w

        @pl.loop(0, chunks)
        def _chunk(g):
            off = base + g * CB
            pltpu.sync_copy(x1_hbm.at[pl.ds(off, CB)], a_v)
            pltpu.sync_copy(x2_hbm.at[pl.ds(off, CB)], b_v)

            @pl.loop(0, CB // _L, unroll=8)
            def _vec(i):
                plsc.addupdate(a_v.at[pl.ds(i * _L, _L)], b_v[pl.ds(i * _L, _L)])

            pltpu.sync_copy(a_v, out_hbm.at[pl.ds(off, CB)])

    return body


def kernel(x_1, x_2):
    M, N = x_1.shape
    out = _sc_add(M * N)(x_1.reshape(-1), x_2.reshape(-1))
    return out.reshape(M, N)


# speedup vs baseline: 1.3746x; 1.3746x over previous
import functools

import jax
import jax.numpy as jnp
from jax import lax
from jax.experimental import pallas as pl
from jax.experimental.pallas import tpu as pltpu
from jax.experimental.pallas import tpu_sc as plsc

_NC, _NS, _L = 2, 16, 16
_NW = _NC * _NS


def _tc_add_body(a_ref, b_ref, o_ref):
    o_ref[...] = a_ref[...] + b_ref[...]


def _tc_add(x_1, x_2, row0):
    M, N = x_1.shape
    BM = 4096
    nb = row0 // BM
    return pl.pallas_call(
        _tc_add_body,
        out_shape=jax.ShapeDtypeStruct((M - row0, N), x_1.dtype),
        grid=((M - row0) // BM,),
        in_specs=[
            pl.BlockSpec((BM, N), lambda i: (i + nb, 0)),
            pl.BlockSpec((BM, N), lambda i: (i + nb, 0)),
        ],
        out_specs=pl.BlockSpec((BM, N), lambda i: (i, 0)),
    )(x_1, x_2)


@functools.cache
def _sc_add(total, interpret=False):
    CB = 32768
    per_w = total // _NW
    chunks = per_w // CB
    mesh = plsc.VectorSubcoreMesh(
        core_axis_name="c", subcore_axis_name="s",
        num_cores=_NC, num_subcores=_NS)

    @functools.partial(
        pl.kernel,
        out_type=jax.ShapeDtypeStruct((total,), jnp.float32),
        mesh=mesh,
        scratch_types=[
            pltpu.VMEM((CB,), jnp.float32),
            pltpu.VMEM((CB,), jnp.float32),
        ],
        interpret=interpret,
    )
    def body(x1_hbm, x2_hbm, out_hbm, a_v, b_v):
        wid = lax.axis_index("s") * _NC + lax.axis_index("c")
        base = wid * per_w

        @pl.loop(0, chunks)
        def _chunk(g):
            off = base + g * CB
            pltpu.sync_copy(x1_hbm.at[pl.ds(off, CB)], a_v)
            pltpu.sync_copy(x2_hbm.at[pl.ds(off, CB)], b_v)

            @pl.loop(0, CB // _L, unroll=8)
            def _vec(i):
                plsc.addupdate(a_v.at[pl.ds(i * _L, _L)], b_v[pl.ds(i * _L, _L)])

            pltpu.sync_copy(a_v, out_hbm.at[pl.ds(off, CB)])

    return body


def kernel(x_1, x_2):
    M, N = x_1.shape
    R = 45056
    sc_out = _sc_add(R * N)(x_1.reshape(-1), x_2.reshape(-1))
    tc_out = _tc_add(x_1, x_2, R)
    return jnp.concatenate([sc_out.reshape(R, N), tc_out], axis=0)


# speedup vs baseline: 1.4635x; 1.0647x over previous
import functools

import jax
import jax.numpy as jnp
from jax import lax
from jax.experimental import pallas as pl
from jax.experimental.pallas import tpu as pltpu
from jax.experimental.pallas import tpu_sc as plsc

_NC, _NS, _L = 2, 16, 16
_NW = _NC * _NS


def _tc_add_body(a_ref, b_ref, o_ref):
    o_ref[...] = a_ref[...] + b_ref[...]


def _tc_add(x_1, x_2, row0):
    M, N = x_1.shape
    BM = 4096
    nb = row0 // BM
    return pl.pallas_call(
        _tc_add_body,
        out_shape=jax.ShapeDtypeStruct((M - row0, N), x_1.dtype),
        grid=((M - row0) // BM,),
        in_specs=[
            pl.BlockSpec((BM, N), lambda i: (i + nb, 0)),
            pl.BlockSpec((BM, N), lambda i: (i + nb, 0)),
        ],
        out_specs=pl.BlockSpec((BM, N), lambda i: (i, 0)),
    )(x_1, x_2)


@functools.cache
def _sc_add(rows, cols):
    CBR = 128
    rows_w = rows // _NW
    chunks = rows_w // CBR
    mesh = plsc.VectorSubcoreMesh(
        core_axis_name="c", subcore_axis_name="s",
        num_cores=_NC, num_subcores=_NS)

    @functools.partial(
        pl.kernel,
        out_type=jax.ShapeDtypeStruct((rows, cols), jnp.float32),
        mesh=mesh,
        scratch_types=[
            pltpu.VMEM((CBR, cols), jnp.float32),
            pltpu.VMEM((CBR, cols), jnp.float32),
        ],
    )
    def body(x1_hbm, x2_hbm, out_hbm, a_v, b_v):
        wid = lax.axis_index("s") * _NC + lax.axis_index("c")
        base = wid * rows_w

        @pl.loop(0, chunks)
        def _chunk(g):
            row0 = base + g * CBR
            pltpu.sync_copy(x1_hbm.at[pl.ds(row0, CBR)], a_v)
            pltpu.sync_copy(x2_hbm.at[pl.ds(row0, CBR)], b_v)

            @pl.loop(0, CBR, unroll=2)
            def _row(r):
                for cc in range(cols // _L):
                    sl = pl.ds(cc * _L, _L)
                    plsc.addupdate(a_v.at[r, sl], b_v[r, sl])

            pltpu.sync_copy(a_v, out_hbm.at[pl.ds(row0, CBR)])

    return body


def kernel(x_1, x_2):
    M, N = x_1.shape
    return _sc_add(M, N)(x_1, x_2)


# speedup vs baseline: 2.7294x; 1.8650x over previous
import functools

import jax
import jax.numpy as jnp
from jax import lax
from jax.experimental import pallas as pl
from jax.experimental.pallas import tpu as pltpu
from jax.experimental.pallas import tpu_sc as plsc

_NC, _NS, _L = 2, 16, 16
_NW = _NC * _NS


def _tc_add_body(a_ref, b_ref, o_ref):
    o_ref[...] = a_ref[...] + b_ref[...]


def _tc_add(x_1, x_2, row0):
    M, N = x_1.shape
    BM = 4096
    nb = row0 // BM
    return pl.pallas_call(
        _tc_add_body,
        out_shape=jax.ShapeDtypeStruct((M - row0, N), x_1.dtype),
        grid=((M - row0) // BM,),
        in_specs=[
            pl.BlockSpec((BM, N), lambda i: (i + nb, 0)),
            pl.BlockSpec((BM, N), lambda i: (i + nb, 0)),
        ],
        out_specs=pl.BlockSpec((BM, N), lambda i: (i, 0)),
    )(x_1, x_2)


@functools.cache
def _sc_add(rows, cols):
    CBR = 128
    rows_w = rows // _NW
    chunks = rows_w // CBR
    mesh = plsc.VectorSubcoreMesh(
        core_axis_name="c", subcore_axis_name="s",
        num_cores=_NC, num_subcores=_NS)

    @functools.partial(
        pl.kernel,
        out_type=jax.ShapeDtypeStruct((rows, cols), jnp.float32),
        mesh=mesh,
        scratch_types=[
            pltpu.VMEM((CBR, cols), jnp.float32),
            pltpu.VMEM((CBR, cols), jnp.float32),
        ],
    )
    def body(x1_hbm, x2_hbm, out_hbm, a_v, b_v):
        wid = lax.axis_index("s") * _NC + lax.axis_index("c")
        base = wid * rows_w

        @pl.loop(0, chunks)
        def _chunk(g):
            row0 = base + g * CBR
            pltpu.sync_copy(x1_hbm.at[pl.ds(row0, CBR)], a_v)
            pltpu.sync_copy(x2_hbm.at[pl.ds(row0, CBR)], b_v)

            @pl.loop(0, CBR, unroll=2)
            def _row(r):
                for cc in range(cols // _L):
                    sl = pl.ds(cc * _L, _L)
                    plsc.addupdate(a_v.at[r, sl], b_v[r, sl])

            pltpu.sync_copy(a_v, out_hbm.at[pl.ds(row0, CBR)])

    return body


def kernel(x_1, x_2):
    M, N = x_1.shape
    R = 61440
    sc_out = _sc_add(R, N)(x_1, x_2)
    tc_out = _tc_add(x_1, x_2, R)
    return jnp.concatenate([sc_out, tc_out], axis=0)


# speedup vs baseline: 4.8272x; 1.7686x over previous
import jax
import jax.numpy as jnp
from jax.experimental import pallas as pl


def _add_body(a_ref, b_ref, o_ref):
    o_ref[...] = a_ref[...] + b_ref[...]


def kernel(x_1, x_2):
    M, N = x_1.shape
    BM = 4096
    return pl.pallas_call(
        _add_body,
        out_shape=jax.ShapeDtypeStruct((M, N), x_1.dtype),
        grid=(M // BM,),
        in_specs=[
            pl.BlockSpec((BM, N), lambda i: (i, 0)),
            pl.BlockSpec((BM, N), lambda i: (i, 0)),
        ],
        out_specs=pl.BlockSpec((BM, N), lambda i: (i, 0)),
    )(x_1, x_2)
